# Initial kernel scaffold; baseline (speedup 1.0000x reference)
import jax
import jax.numpy as jnp
from jax.experimental import pallas as pl


def kernel(x, edge_index, W, attn_l, attn_r):
    raise NotImplementedError("write your pallas kernel here")


# speedup vs baseline: 37.2372x; 37.2372x over previous
import functools

import jax
import jax.numpy as jnp
from jax import lax
from jax.experimental import pallas as pl
from jax.experimental.pallas import tpu as pltpu
from jax.experimental.pallas import tpu_sc as plsc

H = 8
C = 16
ACC_W = H * C + 16

NC = 2
NS = 16
NW = NC * NS
CH = 80


def _take16(v, idx):
    return lax.gather(
        v, idx[:, None],
        dimension_numbers=lax.GatherDimensionNumbers(
            offset_dims=(), collapsed_slice_dims=(0,), start_index_map=(0,)),
        slice_sizes=(1,),
        mode=lax.GatherScatterMode.PROMISE_IN_BOUNDS)


def _mm_body(x_ref, wt_ref, alr_ref, feat_ref, t_ref):
    f = jnp.dot(x_ref[...], wt_ref[...],
                preferred_element_type=jnp.float32,
                precision=lax.Precision.HIGHEST)
    feat_ref[...] = f
    t_ref[...] = jnp.dot(f, alr_ref[...],
                         preferred_element_type=jnp.float32,
                         precision=lax.Precision.HIGHEST)


def _project(x, wt, alr):
    n, d = x.shape
    blk = 2000
    grid = (n // blk,)
    return pl.pallas_call(
        _mm_body,
        grid=grid,
        in_specs=[
            pl.BlockSpec((blk, d), lambda i: (i, 0)),
            pl.BlockSpec((d, d), lambda i: (0, 0)),
            pl.BlockSpec((d, 2 * H), lambda i: (0, 0)),
        ],
        out_specs=[
            pl.BlockSpec((blk, d), lambda i: (i, 0)),
            pl.BlockSpec((blk, 2 * H), lambda i: (i, 0)),
        ],
        out_shape=[
            jax.ShapeDtypeStruct((n, d), jnp.float32),
            jax.ShapeDtypeStruct((n, 2 * H), jnp.float32),
        ],
    )(x, wt, alr)


def _sc_aggregate(feat, t, src, dst, zrows):
    n, d = feat.shape
    e = src.shape[0]
    ew = e // NW
    nch = ew // CH
    npad = ((n + 2047) // 2048) * 2048
    rows_per_sub = npad // NS

    mesh = plsc.VectorSubcoreMesh(core_axis_name="c", subcore_axis_name="s")

    @functools.partial(
        pl.kernel,
        mesh=mesh,
        compiler_params=pltpu.CompilerParams(use_tc_tiling_on_sc=False),
        out_type=jax.ShapeDtypeStruct((NC, npad, ACC_W), jnp.float32),
        scratch_types=[
            pltpu.VMEM((CH,), jnp.int32),
            pltpu.VMEM((CH,), jnp.int32),
            pltpu.VMEM((CH, 2 * H), jnp.float32),
            pltpu.VMEM((CH, 2 * H), jnp.float32),
            pltpu.VMEM((CH, d), jnp.float32),
            pltpu.VMEM((CH, ACC_W), jnp.float32),
            pltpu.VMEM_SHARED((npad, ACC_W), jnp.float32),
        ],
    )
    def sc_gat(feat_hbm, t_hbm, src_hbm, dst_hbm, z_hbm, acc_hbm,
               si, di, ts, td, fb, mb, accsh):
        cidx = lax.axis_index("c")
        sidx = lax.axis_index("s")
        wid = sidx * NC + cidx

        pltpu.sync_copy(z_hbm, accsh.at[pl.ds(sidx * rows_per_sub, rows_per_sub)])
        plsc.subcore_barrier()

        lane = jnp.arange(16, dtype=jnp.int32)
        rot = (lane + 8) & 15
        lane_lt8 = lane < H

        base0 = wid * ew

        @pl.loop(0, nch)
        def _chunk(k):
            base = base0 + k * CH
            pltpu.sync_copy(src_hbm.at[pl.ds(base, CH)], si)
            pltpu.sync_copy(dst_hbm.at[pl.ds(base, CH)], di)
            pltpu.sync_copy(t_hbm.at[si], ts)
            pltpu.sync_copy(t_hbm.at[di], td)
            pltpu.sync_copy(feat_hbm.at[si], fb)

            @pl.loop(0, CH)
            def _edge(i):
                vs = ts[i]
                vd = td[i]
                el_er = vs + _take16(vd, rot)
                lrelu = jnp.where(el_er > 0, el_er, el_er * 0.2)
                w = jnp.exp(lrelu)
                w = jnp.where(lane_lt8, w, 0.0)
                mb[i, pl.ds(H * C, 16)] = w
                for h in range(H):
                    bh = _take16(w, jnp.full((16,), h, jnp.int32))
                    mb[i, pl.ds(h * C, C)] = fb[i, pl.ds(h * C, C)] * bh

            pltpu.sync_copy(mb, accsh.at[di], add=True)

        plsc.subcore_barrier()
        pltpu.sync_copy(accsh.at[pl.ds(sidx * rows_per_sub, rows_per_sub)],
                        acc_hbm.at[cidx, pl.ds(sidx * rows_per_sub, rows_per_sub)])

    return sc_gat(feat, t, src, dst, zrows)


def _combine_body(acc_ref, s_ref, out_ref):
    a = acc_ref[0] + acc_ref[1]
    num = a[:, :H * C]
    den = jnp.dot(a, s_ref[...],
                  preferred_element_type=jnp.float32,
                  precision=lax.Precision.HIGHEST)
    out_ref[...] = num / (den + 1e-9)


def _combine(acc, s):
    _, n, aw = acc.shape
    blk = 1024
    grid = (n // blk,)
    return pl.pallas_call(
        _combine_body,
        grid=grid,
        in_specs=[
            pl.BlockSpec((NC, blk, aw), lambda i: (0, i, 0)),
            pl.BlockSpec((aw, H * C), lambda i: (0, 0)),
        ],
        out_specs=pl.BlockSpec((blk, H * C), lambda i: (i, 0)),
        out_shape=jax.ShapeDtypeStruct((n, H * C), jnp.float32),
    )(acc, s)


def kernel(x, edge_index, W, attn_l, attn_r):
    n, d = x.shape
    wt = W.T
    rep = jnp.repeat(jnp.eye(H, dtype=jnp.float32), C, axis=0)
    alr = jnp.concatenate(
        [rep * attn_l.reshape(-1, 1), rep * attn_r.reshape(-1, 1)], axis=1)

    feat, t = _project(x, wt, alr)

    npad = ((n + 2047) // 2048) * 2048
    zrows = jnp.zeros((npad // NS, ACC_W), jnp.float32)
    acc = _sc_aggregate(feat, t, edge_index[0], edge_index[1], zrows)

    s = jnp.zeros((ACC_W, H * C), jnp.float32).at[H * C:H * C + H, :].set(rep.T)
    out = _combine(acc, s)
    return out[:n].reshape(n, H, C)


# speedup vs baseline: 60.3634x; 1.6211x over previous
import functools

import jax
import jax.numpy as jnp
from jax import lax
from jax.experimental import pallas as pl
from jax.experimental.pallas import tpu as pltpu
from jax.experimental.pallas import tpu_sc as plsc

H = 8
C = 16
ACC_W = H * C + 16

NC = 2
NS = 16
NW = NC * NS
CH = 50


def _take16(v, idx):
    return lax.gather(
        v, idx[:, None],
        dimension_numbers=lax.GatherDimensionNumbers(
            offset_dims=(), collapsed_slice_dims=(0,), start_index_map=(0,)),
        slice_sizes=(1,),
        mode=lax.GatherScatterMode.PROMISE_IN_BOUNDS)


def _mm_body(x_ref, wt_ref, alr_ref, feat_ref, t_ref):
    f = jnp.dot(x_ref[...], wt_ref[...],
                preferred_element_type=jnp.float32,
                precision=lax.Precision.HIGHEST)
    feat_ref[...] = f
    t_ref[...] = jnp.dot(f, alr_ref[...],
                         preferred_element_type=jnp.float32,
                         precision=lax.Precision.HIGHEST)


def _project(x, wt, alr):
    n, d = x.shape
    blk = 2000
    grid = (n // blk,)
    return pl.pallas_call(
        _mm_body,
        grid=grid,
        in_specs=[
            pl.BlockSpec((blk, d), lambda i: (i, 0)),
            pl.BlockSpec((d, d), lambda i: (0, 0)),
            pl.BlockSpec((d, 2 * H), lambda i: (0, 0)),
        ],
        out_specs=[
            pl.BlockSpec((blk, d), lambda i: (i, 0)),
            pl.BlockSpec((blk, 2 * H), lambda i: (i, 0)),
        ],
        out_shape=[
            jax.ShapeDtypeStruct((n, d), jnp.float32),
            jax.ShapeDtypeStruct((n, 2 * H), jnp.float32),
        ],
    )(x, wt, alr)


def _sc_aggregate(feat, t, sd3d, zrows):
    n, d = feat.shape
    _, nch, _, ch = sd3d.shape
    npad = ((n + 2047) // 2048) * 2048
    rows_per_sub = npad // NS
    assert nch % 4 == 0

    mesh = plsc.VectorSubcoreMesh(core_axis_name="c", subcore_axis_name="s")

    @functools.partial(
        pl.kernel,
        mesh=mesh,
        compiler_params=pltpu.CompilerParams(use_tc_tiling_on_sc=False),
        out_type=jax.ShapeDtypeStruct((NC, npad, ACC_W), jnp.float32),
        scratch_types=[
            [pltpu.VMEM((2, ch), jnp.int32)] * 4,
            [pltpu.VMEM((ch, 2 * H), jnp.float32)] * 2,
            [pltpu.VMEM((ch, 2 * H), jnp.float32)] * 2,
            [pltpu.VMEM((ch, d), jnp.float32)] * 2,
            [pltpu.VMEM((ch, ACC_W), jnp.float32)] * 2,
            pltpu.VMEM_SHARED((npad, ACC_W), jnp.float32),
            [pltpu.SemaphoreType.DMA] * 2,
            [pltpu.SemaphoreType.DMA] * 2,
            pltpu.SemaphoreType.DMA,
        ],
    )
    def sc_gat(feat_hbm, t_hbm, sd_hbm, z_hbm, acc_hbm,
               sd4, ts2, td2, fb2, mb2, accsh, gsem2, ssem2, isem):
        cidx = lax.axis_index("c")
        sidx = lax.axis_index("s")
        wid = sidx * NC + cidx

        pltpu.sync_copy(z_hbm, accsh.at[pl.ds(sidx * rows_per_sub, rows_per_sub)])
        pltpu.sync_copy(sd_hbm.at[wid, 0], sd4[0])
        pltpu.sync_copy(sd_hbm.at[wid, 1], sd4[1])
        plsc.subcore_barrier()

        lane = jnp.arange(16, dtype=jnp.int32)
        rot = (lane + 8) & 15
        lane_lt8 = lane < H

        def start_gathers(b, sd):
            pltpu.async_copy(t_hbm.at[sd.at[0]], ts2[b], gsem2[b])
            pltpu.async_copy(t_hbm.at[sd.at[1]], td2[b], gsem2[b])
            pltpu.async_copy(feat_hbm.at[sd.at[0]], fb2[b], gsem2[b])

        def wait_gathers(b, sd):
            pltpu.make_async_copy(t_hbm.at[sd.at[0]], ts2[b], gsem2[b]).wait()
            pltpu.make_async_copy(t_hbm.at[sd.at[1]], td2[b], gsem2[b]).wait()
            pltpu.make_async_copy(feat_hbm.at[sd.at[0]], fb2[b], gsem2[b]).wait()

        def drain_scatter(b, sd):
            pltpu.make_async_copy(mb2[b], accsh.at[sd.at[1]], ssem2[b]).wait()

        def compute(b, sd):
            ts, td, fb, mb = ts2[b], td2[b], fb2[b], mb2[b]

            @pl.loop(0, ch)
            def _edge(i):
                vs = ts[i]
                vd = td[i]
                el_er = vs + _take16(vd, rot)
                lrelu = jnp.where(el_er > 0, el_er, el_er * 0.2)
                w = jnp.exp(lrelu)
                w = jnp.where(lane_lt8, w, 0.0)
                mb[i, pl.ds(H * C, 16)] = w
                for h in range(H):
                    bh = _take16(w, jnp.full((16,), h, jnp.int32))
                    mb[i, pl.ds(h * C, C)] = fb[i, pl.ds(h * C, C)] * bh

            pltpu.async_copy(mb, accsh.at[sd.at[1]], ssem2[b], add=True)

        start_gathers(0, sd4[0])

        @pl.loop(0, nch // 4)
        def _quad(jj):
            for s in range(4):
                k = 4 * jj + s
                b = s % 2

                @pl.when(k >= 2)
                def _():
                    drain_scatter(b, sd4[(s + 2) % 4])

                @pl.when(k + 2 < nch)
                def _():
                    pltpu.async_copy(sd_hbm.at[wid, k + 2], sd4[(s + 2) % 4],
                                     isem)

                @pl.when((k + 1 >= 2) & (k + 1 < nch))
                def _():
                    pltpu.make_async_copy(sd_hbm.at[wid, k + 1],
                                          sd4[(s + 1) % 4], isem).wait()

                @pl.when(k + 1 < nch)
                def _():
                    start_gathers(1 - b, sd4[(s + 1) % 4])

                wait_gathers(b, sd4[s])
                compute(b, sd4[s])

        drain_scatter(nch % 2, sd4[(nch - 2) % 4])
        drain_scatter(1 - nch % 2, sd4[(nch - 1) % 4])
        plsc.subcore_barrier()
        pltpu.sync_copy(accsh.at[pl.ds(sidx * rows_per_sub, rows_per_sub)],
                        acc_hbm.at[cidx, pl.ds(sidx * rows_per_sub, rows_per_sub)])

    return sc_gat(feat, t, sd3d, zrows)


def _combine_body(acc_ref, s_ref, out_ref):
    a = acc_ref[0] + acc_ref[1]
    num = a[:, :H * C]
    den = jnp.dot(a, s_ref[...],
                  preferred_element_type=jnp.float32,
                  precision=lax.Precision.HIGHEST)
    out_ref[...] = num / (den + 1e-9)


def _combine(acc, s):
    _, n, aw = acc.shape
    blk = 1024
    grid = (n // blk,)
    return pl.pallas_call(
        _combine_body,
        grid=grid,
        in_specs=[
            pl.BlockSpec((NC, blk, aw), lambda i: (0, i, 0)),
            pl.BlockSpec((aw, H * C), lambda i: (0, 0)),
        ],
        out_specs=pl.BlockSpec((blk, H * C), lambda i: (i, 0)),
        out_shape=jax.ShapeDtypeStruct((n, H * C), jnp.float32),
    )(acc, s)


def kernel(x, edge_index, W, attn_l, attn_r):
    n, d = x.shape
    wt = W.T
    rep = jnp.repeat(jnp.eye(H, dtype=jnp.float32), C, axis=0)
    alr = jnp.concatenate(
        [rep * attn_l.reshape(-1, 1), rep * attn_r.reshape(-1, 1)], axis=1)

    feat, t = _project(x, wt, alr)

    npad = ((n + 2047) // 2048) * 2048
    zrows = jnp.zeros((npad // NS, ACC_W), jnp.float32)
    e = edge_index.shape[1]
    ew = e // NW
    sd3d = edge_index.reshape(2, NW, ew // CH, CH).transpose(1, 2, 0, 3)
    acc = _sc_aggregate(feat, t, sd3d, zrows)

    s = jnp.zeros((ACC_W, H * C), jnp.float32).at[H * C:H * C + H, :].set(rep.T)
    out = _combine(acc, s)
    return out[:n].reshape(n, H, C)
